# Initial kernel scaffold; baseline (speedup 1.0000x reference)
import functools

import jax
import jax.numpy as jnp
from jax import lax
from jax.experimental import pallas as pl
from jax.experimental.pallas import tpu as pltpu
from jax.experimental.pallas import tpu_sc as plsc

N = 100000
E = 3200000
H = 64
CHUNK = 25600
NCH = 4
NP = CHUNK * NCH
B = 1024
SEG = 128
NSEG = B // SEG
NBLK = 196
EP = 16 * B * NBLK
EPAD = EP - E
EROWS = EP // 128
DEG_NBLK = 98
BR = 2048
NBR = NP // BR
F32 = jnp.float32


def _mesh():
    return plsc.VectorSubcoreMesh(
        core_axis_name="c", subcore_axis_name="s", num_cores=2, num_subcores=16
    )


def _deg_body(dst2d, zeros1, out, dstv, onesv, zb1, stage, acc1, sem):
    c = lax.axis_index("c")
    s = lax.axis_index("s")
    w = c * 16 + s
    pltpu.sync_copy(zeros1, zb1)
    for k in range(8):
        onesv[pl.ds(k * 16, 16)] = jnp.ones((16,), F32)
    base = s * 6400
    for k in range(6):
        pltpu.sync_copy(zb1, acc1.at[pl.ds(base + k * 1024, 1024)])
    pltpu.sync_copy(zb1.at[pl.ds(0, 256)], acc1.at[pl.ds(base + 6144, 256)])
    plsc.subcore_barrier()

    def blk(b, carry):
        roff = w * (DEG_NBLK * 8) + b * 8
        pltpu.sync_copy(dst2d.at[pl.ds(roff, 8)], dstv)
        cps = [
            pltpu.async_copy(onesv, acc1.at[dstv.at[i]], sem, add=True)
            for i in range(8)
        ]
        for cp in cps:
            cp.wait()
        return carry

    lax.fori_loop(0, DEG_NBLK, blk, 0)
    plsc.subcore_barrier()
    obase = c * NP + s * 6400
    for k in range(6):
        pltpu.sync_copy(acc1.at[pl.ds(base + k * 1024, 1024)], stage)
        pltpu.sync_copy(stage, out.at[pl.ds(obase + k * 1024, 1024)])
    pltpu.sync_copy(acc1.at[pl.ds(base + 6144, 256)], stage.at[pl.ds(0, 256)])
    pltpu.sync_copy(stage.at[pl.ds(0, 256)], out.at[pl.ds(obase + 6144, 256)])


_deg_call = pl.kernel(
    _deg_body,
    out_type=jax.ShapeDtypeStruct((2 * NP,), F32),
    mesh=_mesh(),
    scratch_types=[
        pltpu.VMEM((8, 128), jnp.int32),
        pltpu.VMEM((128,), F32),
        pltpu.VMEM((1024,), F32),
        pltpu.VMEM((1024,), F32),
        pltpu.VMEM_SHARED((NP + 16,), F32),
        pltpu.SemaphoreType.DMA,
    ],
)


def _seg_body(src_hbm, dst2d, g_hbm, zrows, out, srcv, dstv, ldstv, rows, zb,
              acc, sem):
    c = lax.axis_index("c")
    s = lax.axis_index("s")
    pltpu.sync_copy(zrows, zb)

    def rnd(r, carry):
        lo = (2 * r + c) * CHUNK
        zbase = s * 1600
        for k in range(12):
            pltpu.sync_copy(zb, acc.at[pl.ds(zbase + k * 128, 128)])
        pltpu.sync_copy(zb.at[pl.ds(0, 64)], acc.at[pl.ds(zbase + 1536, 64)])
        plsc.subcore_barrier()
        dummy = CHUNK + s

        def blk(b, carry2):
            roff = s * (NBLK * 8) + b * 8
            eoff = roff * 128
            pltpu.sync_copy(src_hbm.at[pl.ds(eoff, B)], srcv)
            pltpu.sync_copy(dst2d.at[pl.ds(roff, 8)], dstv)
            gcps = [
                pltpu.async_copy(
                    g_hbm.at[srcv.at[pl.ds(i * SEG, SEG)]],
                    rows.at[pl.ds(i * SEG, SEG)],
                    sem,
                )
                for i in range(NSEG)
            ]
            for i in range(8):
                def sel(j, carry3):
                    d = dstv[i, pl.ds(j * 16, 16)]
                    m = (d >= lo) & (d < lo + CHUNK)
                    ldstv[i, pl.ds(j * 16, 16)] = jnp.where(m, d - lo, dummy)
                    return carry3
                lax.fori_loop(0, 8, sel, 0)
            for cp in gcps:
                cp.wait()
            scps = [
                pltpu.async_copy(
                    rows.at[pl.ds(i * SEG, SEG)],
                    acc.at[ldstv.at[i]],
                    sem,
                    add=True,
                )
                for i in range(NSEG)
            ]
            for cp in scps:
                cp.wait()
            return carry2

        lax.fori_loop(0, NBLK, blk, 0)
        plsc.subcore_barrier()
        wb = s * 1600
        pltpu.sync_copy(acc.at[pl.ds(wb, 1024)], rows)
        pltpu.sync_copy(rows, out.at[pl.ds(lo + wb, 1024)])
        pltpu.sync_copy(acc.at[pl.ds(wb + 1024, 576)], rows.at[pl.ds(0, 576)])
        pltpu.sync_copy(rows.at[pl.ds(0, 576)], out.at[pl.ds(lo + wb + 1024, 576)])
        plsc.subcore_barrier()
        return carry

    lax.fori_loop(0, 2, rnd, 0)


_seg_call = pl.kernel(
    _seg_body,
    out_type=jax.ShapeDtypeStruct((NP, H), F32),
    mesh=_mesh(),
    scratch_types=[
        pltpu.VMEM((B,), jnp.int32),
        pltpu.VMEM((8, 128), jnp.int32),
        pltpu.VMEM((8, 128), jnp.int32),
        pltpu.VMEM((B, H), F32),
        pltpu.VMEM((128, H), F32),
        pltpu.VMEM_SHARED((CHUNK + 16, H), F32),
        pltpu.SemaphoreType.DMA,
    ],
)


def _tc1_body(x_ref, deg_ref, w1_ref, dinv_ref, g1_ref):
    deg = jnp.sum(deg_ref[...], axis=1, keepdims=True) + 1.0
    dinv = lax.rsqrt(deg)
    hw = jnp.dot(x_ref[...], w1_ref[...], preferred_element_type=F32)
    dinv_ref[...] = dinv
    g1_ref[...] = hw * dinv


def _tc1(x_p, degT, w1p):
    return pl.pallas_call(
        _tc1_body,
        grid=(NBR,),
        in_specs=[
            pl.BlockSpec((BR, 8), lambda i: (i, 0)),
            pl.BlockSpec((BR, 8), lambda i: (i, 0)),
            pl.BlockSpec((8, H), lambda i: (0, 0)),
        ],
        out_specs=[
            pl.BlockSpec((BR, 1), lambda i: (i, 0)),
            pl.BlockSpec((BR, H), lambda i: (i, 0)),
        ],
        out_shape=[
            jax.ShapeDtypeStruct((NP, 1), F32),
            jax.ShapeDtypeStruct((NP, H), F32),
        ],
    )(x_p, degT, w1p)


def _tc2_body(s1_ref, g1_ref, dinv_ref, w2_ref, b1_ref, g2_ref):
    dinv = dinv_ref[...]
    h1 = jnp.maximum(dinv * (s1_ref[...] + g1_ref[...]) + b1_ref[...], 0.0)
    g2_ref[...] = jnp.dot(h1, w2_ref[...], preferred_element_type=F32) * dinv


def _tc2(s1, g1, dinv, w2, b1r):
    return pl.pallas_call(
        _tc2_body,
        grid=(NBR,),
        in_specs=[
            pl.BlockSpec((BR, H), lambda i: (i, 0)),
            pl.BlockSpec((BR, H), lambda i: (i, 0)),
            pl.BlockSpec((BR, 1), lambda i: (i, 0)),
            pl.BlockSpec((H, H), lambda i: (0, 0)),
            pl.BlockSpec((1, H), lambda i: (0, 0)),
        ],
        out_specs=pl.BlockSpec((BR, H), lambda i: (i, 0)),
        out_shape=jax.ShapeDtypeStruct((NP, H), F32),
    )(s1, g1, dinv, w2, b1r)


def _tc3_body(s2_ref, g2_ref, dinv_ref, x_ref, b2_ref, wp1a_ref, wp1b_ref,
              bp1_ref, wp2_ref, bp2_ref, out_ref, acc_h, acc_st):
    i = pl.program_id(0)

    @pl.when(i == 0)
    def _init():
        acc_h[...] = jnp.zeros((1, H), F32)
        acc_st[...] = jnp.zeros((1, 8), F32)

    rowid = lax.broadcasted_iota(jnp.int32, (BR, 1), 0) + i * BR
    valid = (rowid < N).astype(F32)
    h2 = jnp.maximum(dinv_ref[...] * (s2_ref[...] + g2_ref[...]) + b2_ref[...],
                     0.0) * valid
    acc_h[...] += jnp.sum(h2, axis=0, keepdims=True)

    xb = x_ref[...] * valid
    x0 = xb[:, 0:1]
    x1 = xb[:, 1:2]
    x2 = xb[:, 2:3]
    x3 = xb[:, 3:4]
    x4 = xb[:, 4:5]
    m = jnp.where((x2 == 1.0) & (valid > 0.0), 1.0, 0.0)
    st = jnp.concatenate(
        [
            jnp.sum(x2, axis=0, keepdims=True),
            jnp.sum(x3, axis=0, keepdims=True),
            jnp.sum(x4, axis=0, keepdims=True),
            jnp.sum(x0 * m, axis=0, keepdims=True),
            jnp.sum(x1 * m, axis=0, keepdims=True),
            jnp.sum(m, axis=0, keepdims=True),
            jnp.zeros((1, 2), F32),
        ],
        axis=1,
    )
    acc_st[...] += st

    @pl.when(i == NBR - 1)
    def _fin():
        e = acc_h[...] * (1.0 / N)
        stv = acc_st[...]
        n_comp = stv[0, 0]
        n_and = stv[0, 1]
        n_or = stv[0, 2]
        sl = stv[0, 3]
        sm = stv[0, 4]
        cnt = stv[0, 5]
        has = cnt > 0.0
        cntc = jnp.maximum(cnt, 1.0)
        avg_l = jnp.where(has, sl / cntc, 0.0)
        avg_m = jnp.where(has, sm / cntc, 0.0)
        gf = jnp.concatenate(
            [
                n_comp.reshape(1, 1),
                n_and.reshape(1, 1),
                n_or.reshape(1, 1),
                (n_and + n_or).reshape(1, 1),
                avg_l.reshape(1, 1),
                avg_m.reshape(1, 1),
                jnp.full((1, 1), 0.2, F32),
                jnp.zeros((1, 1), F32),
            ],
            axis=1,
        )
        pre = (
            jnp.dot(e, wp1a_ref[...], preferred_element_type=F32)
            + jnp.dot(gf, wp1b_ref[...], preferred_element_type=F32)
            + bp1_ref[...]
        )
        hid = jnp.maximum(pre, 0.0)
        raw = jnp.dot(hid, wp2_ref[...], preferred_element_type=F32) + bp2_ref[...]
        out_ref[...] = 2.0 + jax.nn.sigmoid(raw) * 4.0


def _tc3(s2, g2, dinv, x_p, b2r, wp1a, wp1b, bp1r, wp2, bp2r):
    return pl.pallas_call(
        _tc3_body,
        grid=(NBR,),
        in_specs=[
            pl.BlockSpec((BR, H), lambda i: (i, 0)),
            pl.BlockSpec((BR, H), lambda i: (i, 0)),
            pl.BlockSpec((BR, 1), lambda i: (i, 0)),
            pl.BlockSpec((BR, 8), lambda i: (i, 0)),
            pl.BlockSpec((1, H), lambda i: (0, 0)),
            pl.BlockSpec((H, 32), lambda i: (0, 0)),
            pl.BlockSpec((8, 32), lambda i: (0, 0)),
            pl.BlockSpec((1, 32), lambda i: (0, 0)),
            pl.BlockSpec((32, 2), lambda i: (0, 0)),
            pl.BlockSpec((1, 2), lambda i: (0, 0)),
        ],
        out_specs=pl.BlockSpec((1, 2), lambda i: (0, 0)),
        out_shape=jax.ShapeDtypeStruct((1, 2), F32),
        scratch_shapes=[pltpu.VMEM((1, H), F32), pltpu.VMEM((1, 8), F32)],
    )(s2, g2, dinv, x_p, b2r, wp1a, wp1b, bp1r, wp2, bp2r)


def kernel(x, edge_index, W1, b1, W2, b2, Wp1, bp1, Wp2, bp2):
    src = edge_index[0].astype(jnp.int32)
    dst = edge_index[1].astype(jnp.int32)
    pad_src = jnp.arange(EPAD, dtype=jnp.int32) % N
    pad_dst = jnp.full((EPAD,), NP, jnp.int32)
    src_p = jnp.concatenate([src, pad_src])
    dst2d = jnp.concatenate([dst, pad_dst]).reshape(EROWS, 128)
    zeros1 = jnp.zeros((1024,), F32)
    zrows = jnp.zeros((128, H), F32)

    deg_flat = _deg_call(dst2d, zeros1)
    degT = jnp.pad(deg_flat.reshape(2, NP).T, ((0, 0), (0, 6)))

    x_p = jnp.pad(x.astype(F32), ((0, NP - N), (0, 3)))
    w1p = jnp.pad(W1.astype(F32), ((0, 3), (0, 0)))
    dinv, g1 = _tc1(x_p, degT, w1p)

    s1 = _seg_call(src_p, dst2d, g1, zrows)
    g2 = _tc2(s1, g1, dinv, W2.astype(F32), b1.reshape(1, H).astype(F32))
    s2 = _seg_call(src_p, dst2d, g2, zrows)

    wp1 = Wp1.astype(F32)
    wp1a = wp1[:H]
    wp1b = jnp.pad(wp1[H:], ((0, 1), (0, 0)))
    out = _tc3(
        s2, g2, dinv, x_p,
        b2.reshape(1, H).astype(F32),
        wp1a, wp1b,
        bp1.reshape(1, 32).astype(F32),
        Wp2.astype(F32),
        bp2.reshape(1, 2).astype(F32),
    )
    return out


# speedup vs baseline: 2.4650x; 2.4650x over previous
import functools

import jax
import jax.numpy as jnp
from jax import lax
from jax.experimental import pallas as pl
from jax.experimental.pallas import tpu as pltpu
from jax.experimental.pallas import tpu_sc as plsc

N = 100000
E = 3200000
H = 64
W = 128
CHUNK = 10240
NCH = 10
NR = NCH // 2
NP = CHUNK * NCH
B = 256
SEG = 128
NSEG = B // SEG
NBLK = 784
EP = 16 * B * NBLK
EPAD = EP - E
EROWS = EP // 128
DEG_NBLK = 98
RPT = CHUNK // 16
BR = 2048
NBR = NP // BR
F32 = jnp.float32


def _mesh():
    return plsc.VectorSubcoreMesh(
        core_axis_name="c", subcore_axis_name="s", num_cores=2, num_subcores=16
    )


def _deg_body(dst2d, zeros1, out, dstv, onesv, zb1, stage, acc1, sem):
    c = lax.axis_index("c")
    s = lax.axis_index("s")
    w = c * 16 + s
    pltpu.sync_copy(zeros1, zb1)
    for k in range(8):
        onesv[pl.ds(k * 16, 16)] = jnp.ones((16,), F32)
    base = s * 6400
    for k in range(6):
        pltpu.sync_copy(zb1, acc1.at[pl.ds(base + k * 1024, 1024)])
    pltpu.sync_copy(zb1.at[pl.ds(0, 256)], acc1.at[pl.ds(base + 6144, 256)])
    plsc.subcore_barrier()

    def blk(b, carry):
        roff = w * (DEG_NBLK * 8) + b * 8
        pltpu.sync_copy(dst2d.at[pl.ds(roff, 8)], dstv)
        cps = [
            pltpu.async_copy(onesv, acc1.at[dstv.at[i]], sem, add=True)
            for i in range(8)
        ]
        for cp in cps:
            cp.wait()
        return carry

    lax.fori_loop(0, DEG_NBLK, blk, 0)
    plsc.subcore_barrier()
    obase = c * NP + s * 6400
    for k in range(6):
        pltpu.sync_copy(acc1.at[pl.ds(base + k * 1024, 1024)], stage)
        pltpu.sync_copy(stage, out.at[pl.ds(obase + k * 1024, 1024)])
    pltpu.sync_copy(acc1.at[pl.ds(base + 6144, 256)], stage.at[pl.ds(0, 256)])
    pltpu.sync_copy(stage.at[pl.ds(0, 256)], out.at[pl.ds(obase + 6144, 256)])


_deg_call = pl.kernel(
    _deg_body,
    out_type=jax.ShapeDtypeStruct((2 * NP,), F32),
    mesh=_mesh(),
    scratch_types=[
        pltpu.VMEM((8, 128), jnp.int32),
        pltpu.VMEM((128,), F32),
        pltpu.VMEM((1024,), F32),
        pltpu.VMEM((1024,), F32),
        pltpu.VMEM_SHARED((NP + 16,), F32),
        pltpu.SemaphoreType.DMA,
    ],
)


def _seg_body(src_hbm, dst2d, g_hbm, zrows, out, srcv, dstv, ldstv, rows, zb,
              acc, sem):
    c = lax.axis_index("c")
    s = lax.axis_index("s")
    pltpu.sync_copy(zrows, zb)

    def rnd(r, carry):
        lo = (2 * r + c) * CHUNK
        zbase = s * RPT
        for k in range(20):
            pltpu.sync_copy(zb, acc.at[pl.ds(zbase + k * 32, 32)])
        plsc.subcore_barrier()
        dummy = CHUNK + s

        def blk(b, carry2):
            roff = s * (NBLK * 2) + b * 2
            eoff = roff * 128
            pltpu.sync_copy(src_hbm.at[pl.ds(eoff, B)], srcv)
            pltpu.sync_copy(dst2d.at[pl.ds(roff, 2)], dstv)
            gcps = [
                pltpu.async_copy(
                    g_hbm.at[srcv.at[pl.ds(i * SEG, SEG)]],
                    rows.at[pl.ds(i * SEG, SEG)],
                    sem,
                )
                for i in range(NSEG)
            ]
            for i in range(NSEG):
                def sel(j, carry3):
                    d = dstv[i, pl.ds(j * 16, 16)]
                    m = (d >= lo) & (d < lo + CHUNK)
                    ldstv[i, pl.ds(j * 16, 16)] = jnp.where(m, d - lo, dummy)
                    return carry3
                lax.fori_loop(0, 8, sel, 0)
            for cp in gcps:
                cp.wait()
            scps = [
                pltpu.async_copy(
                    rows.at[pl.ds(i * SEG, SEG)],
                    acc.at[ldstv.at[i]],
                    sem,
                    add=True,
                )
                for i in range(NSEG)
            ]
            for cp in scps:
                cp.wait()
            return carry2

        lax.fori_loop(0, NBLK, blk, 0)
        plsc.subcore_barrier()
        wb = s * RPT
        for k in range(2):
            pltpu.sync_copy(acc.at[pl.ds(wb + k * 256, 256)], rows)
            pltpu.sync_copy(rows, out.at[pl.ds(lo + wb + k * 256, 256)])
        pltpu.sync_copy(acc.at[pl.ds(wb + 512, 128)], rows.at[pl.ds(0, 128)])
        pltpu.sync_copy(rows.at[pl.ds(0, 128)], out.at[pl.ds(lo + wb + 512, 128)])
        plsc.subcore_barrier()
        return carry

    lax.fori_loop(0, NR, rnd, 0)


_seg_call = pl.kernel(
    _seg_body,
    out_type=jax.ShapeDtypeStruct((NP, W), F32),
    mesh=_mesh(),
    scratch_types=[
        pltpu.VMEM((B,), jnp.int32),
        pltpu.VMEM((NSEG, 128), jnp.int32),
        pltpu.VMEM((NSEG, 128), jnp.int32),
        pltpu.VMEM((B, W), F32),
        pltpu.VMEM((32, W), F32),
        pltpu.VMEM_SHARED((CHUNK + 16, W), F32),
        pltpu.SemaphoreType.DMA,
    ],
)


def _tc1_body(x_ref, deg_ref, w1_ref, dinv_ref, g1_ref):
    deg = jnp.sum(deg_ref[...], axis=1, keepdims=True) + 1.0
    dinv = lax.rsqrt(deg)
    hw = jnp.dot(x_ref[...], w1_ref[...], preferred_element_type=F32)
    dinv_ref[...] = dinv
    g1_ref[...] = jnp.concatenate(
        [hw * dinv, jnp.zeros((BR, W - H), F32)], axis=1)


def _tc1(x_p, degT, w1p):
    return pl.pallas_call(
        _tc1_body,
        grid=(NBR,),
        in_specs=[
            pl.BlockSpec((BR, 8), lambda i: (i, 0)),
            pl.BlockSpec((BR, 8), lambda i: (i, 0)),
            pl.BlockSpec((8, H), lambda i: (0, 0)),
        ],
        out_specs=[
            pl.BlockSpec((BR, 1), lambda i: (i, 0)),
            pl.BlockSpec((BR, W), lambda i: (i, 0)),
        ],
        out_shape=[
            jax.ShapeDtypeStruct((NP, 1), F32),
            jax.ShapeDtypeStruct((NP, W), F32),
        ],
    )(x_p, degT, w1p)


def _tc2_body(s1_ref, g1_ref, dinv_ref, w2_ref, b1_ref, g2_ref):
    dinv = dinv_ref[...]
    h1 = jnp.maximum(
        dinv * (s1_ref[:, :H] + g1_ref[:, :H]) + b1_ref[...], 0.0)
    g2 = jnp.dot(h1, w2_ref[...], preferred_element_type=F32) * dinv
    g2_ref[...] = jnp.concatenate([g2, jnp.zeros((BR, W - H), F32)], axis=1)


def _tc2(s1, g1, dinv, w2, b1r):
    return pl.pallas_call(
        _tc2_body,
        grid=(NBR,),
        in_specs=[
            pl.BlockSpec((BR, W), lambda i: (i, 0)),
            pl.BlockSpec((BR, W), lambda i: (i, 0)),
            pl.BlockSpec((BR, 1), lambda i: (i, 0)),
            pl.BlockSpec((H, H), lambda i: (0, 0)),
            pl.BlockSpec((1, H), lambda i: (0, 0)),
        ],
        out_specs=pl.BlockSpec((BR, W), lambda i: (i, 0)),
        out_shape=jax.ShapeDtypeStruct((NP, W), F32),
    )(s1, g1, dinv, w2, b1r)


def _tc3_body(s2_ref, g2_ref, dinv_ref, x_ref, b2_ref, wp1a_ref, wp1b_ref,
              bp1_ref, wp2_ref, bp2_ref, out_ref, acc_h, acc_st):
    i = pl.program_id(0)

    @pl.when(i == 0)
    def _init():
        acc_h[...] = jnp.zeros((1, H), F32)
        acc_st[...] = jnp.zeros((1, 8), F32)

    rowid = lax.broadcasted_iota(jnp.int32, (BR, 1), 0) + i * BR
    valid = (rowid < N).astype(F32)
    h2 = jnp.maximum(
        dinv_ref[...] * (s2_ref[:, :H] + g2_ref[:, :H]) + b2_ref[...],
        0.0) * valid
    acc_h[...] += jnp.sum(h2, axis=0, keepdims=True)

    xb = x_ref[...] * valid
    x0 = xb[:, 0:1]
    x1 = xb[:, 1:2]
    x2 = xb[:, 2:3]
    x3 = xb[:, 3:4]
    x4 = xb[:, 4:5]
    m = jnp.where((x2 == 1.0) & (valid > 0.0), 1.0, 0.0)
    st = jnp.concatenate(
        [
            jnp.sum(x2, axis=0, keepdims=True),
            jnp.sum(x3, axis=0, keepdims=True),
            jnp.sum(x4, axis=0, keepdims=True),
            jnp.sum(x0 * m, axis=0, keepdims=True),
            jnp.sum(x1 * m, axis=0, keepdims=True),
            jnp.sum(m, axis=0, keepdims=True),
            jnp.zeros((1, 2), F32),
        ],
        axis=1,
    )
    acc_st[...] += st

    @pl.when(i == NBR - 1)
    def _fin():
        e = acc_h[...] * (1.0 / N)
        stv = acc_st[...]
        n_comp = stv[0, 0]
        n_and = stv[0, 1]
        n_or = stv[0, 2]
        sl = stv[0, 3]
        sm = stv[0, 4]
        cnt = stv[0, 5]
        has = cnt > 0.0
        cntc = jnp.maximum(cnt, 1.0)
        avg_l = jnp.where(has, sl / cntc, 0.0)
        avg_m = jnp.where(has, sm / cntc, 0.0)
        gf = jnp.concatenate(
            [
                n_comp.reshape(1, 1),
                n_and.reshape(1, 1),
                n_or.reshape(1, 1),
                (n_and + n_or).reshape(1, 1),
                avg_l.reshape(1, 1),
                avg_m.reshape(1, 1),
                jnp.full((1, 1), 0.2, F32),
                jnp.zeros((1, 1), F32),
            ],
            axis=1,
        )
        pre = (
            jnp.dot(e, wp1a_ref[...], preferred_element_type=F32)
            + jnp.dot(gf, wp1b_ref[...], preferred_element_type=F32)
            + bp1_ref[...]
        )
        hid = jnp.maximum(pre, 0.0)
        raw = jnp.dot(hid, wp2_ref[...], preferred_element_type=F32) + bp2_ref[...]
        out_ref[...] = 2.0 + jax.nn.sigmoid(raw) * 4.0


def _tc3(s2, g2, dinv, x_p, b2r, wp1a, wp1b, bp1r, wp2, bp2r):
    return pl.pallas_call(
        _tc3_body,
        grid=(NBR,),
        in_specs=[
            pl.BlockSpec((BR, W), lambda i: (i, 0)),
            pl.BlockSpec((BR, W), lambda i: (i, 0)),
            pl.BlockSpec((BR, 1), lambda i: (i, 0)),
            pl.BlockSpec((BR, 8), lambda i: (i, 0)),
            pl.BlockSpec((1, H), lambda i: (0, 0)),
            pl.BlockSpec((H, 32), lambda i: (0, 0)),
            pl.BlockSpec((8, 32), lambda i: (0, 0)),
            pl.BlockSpec((1, 32), lambda i: (0, 0)),
            pl.BlockSpec((32, 2), lambda i: (0, 0)),
            pl.BlockSpec((1, 2), lambda i: (0, 0)),
        ],
        out_specs=pl.BlockSpec((1, 2), lambda i: (0, 0)),
        out_shape=jax.ShapeDtypeStruct((1, 2), F32),
        scratch_shapes=[pltpu.VMEM((1, H), F32), pltpu.VMEM((1, 8), F32)],
    )(s2, g2, dinv, x_p, b2r, wp1a, wp1b, bp1r, wp2, bp2r)


def kernel(x, edge_index, W1, b1, W2, b2, Wp1, bp1, Wp2, bp2):
    src = edge_index[0].astype(jnp.int32)
    dst = edge_index[1].astype(jnp.int32)
    pad_src = jnp.arange(EPAD, dtype=jnp.int32) % N
    pad_dst = jnp.full((EPAD,), NP, jnp.int32)
    src_p = jnp.concatenate([src, pad_src])
    dst2d = jnp.concatenate([dst, pad_dst]).reshape(EROWS, 128)
    zeros1 = jnp.zeros((1024,), F32)
    zrows = jnp.zeros((32, W), F32)

    deg_flat = _deg_call(dst2d, zeros1)
    degT = jnp.pad(deg_flat.reshape(2, NP).T, ((0, 0), (0, 6)))

    x_p = jnp.pad(x.astype(F32), ((0, NP - N), (0, 3)))
    w1p = jnp.pad(W1.astype(F32), ((0, 3), (0, 0)))
    dinv, g1 = _tc1(x_p, degT, w1p)

    s1 = _seg_call(src_p, dst2d, g1, zrows)
    g2 = _tc2(s1, g1, dinv, W2.astype(F32), b1.reshape(1, H).astype(F32))
    s2 = _seg_call(src_p, dst2d, g2, zrows)

    wp1 = Wp1.astype(F32)
    wp1a = wp1[:H]
    wp1b = jnp.pad(wp1[H:], ((0, 1), (0, 0)))
    out = _tc3(
        s2, g2, dinv, x_p,
        b2.reshape(1, H).astype(F32),
        wp1a, wp1b,
        bp1.reshape(1, 32).astype(F32),
        Wp2.astype(F32),
        bp2.reshape(1, 2).astype(F32),
    )
    return out


# speedup vs baseline: 11.0735x; 4.4923x over previous
import functools

import jax


def _dyn_gather(x, idx):
    dnums = jax.lax.GatherDimensionNumbers(
        offset_dims=(), collapsed_slice_dims=(0,), start_index_map=(0,))
    return jax.lax.gather(
        x, idx[:, None], dnums, slice_sizes=(1,),
        mode=jax.lax.GatherScatterMode.PROMISE_IN_BOUNDS)
import jax.numpy as jnp
from jax import lax
from jax.experimental import pallas as pl
from jax.experimental.pallas import tpu as pltpu
from jax.experimental.pallas import tpu_sc as plsc

N = 100000
E = 3200000
H = 64
W = 128
CHUNK = 10240
NCH = 10
NR = NCH // 2
NP = CHUNK * NCH
B = 2048
SEG = 128
NSEG = B // SEG
NBLK = 98
EP = 16 * B * NBLK
EPAD = EP - E
EROWS = EP // 128
DEG_NBLK = 98
RPT = CHUNK // 16
BR = 2048
NBR = NP // BR
F32 = jnp.float32


def _mesh():
    return plsc.VectorSubcoreMesh(
        core_axis_name="c", subcore_axis_name="s", num_cores=2, num_subcores=16
    )


def _deg_body(dst2d, zeros1, out, dstv, onesv, zb1, stage, acc1, sem):
    c = lax.axis_index("c")
    s = lax.axis_index("s")
    w = c * 16 + s
    pltpu.sync_copy(zeros1, zb1)
    for k in range(8):
        onesv[pl.ds(k * 16, 16)] = jnp.ones((16,), F32)
    base = s * 6400
    for k in range(6):
        pltpu.sync_copy(zb1, acc1.at[pl.ds(base + k * 1024, 1024)])
    pltpu.sync_copy(zb1.at[pl.ds(0, 256)], acc1.at[pl.ds(base + 6144, 256)])
    plsc.subcore_barrier()

    def blk(b, carry):
        roff = w * (DEG_NBLK * 8) + b * 8
        pltpu.sync_copy(dst2d.at[pl.ds(roff, 8)], dstv)
        cps = [
            pltpu.async_copy(onesv, acc1.at[dstv.at[i]], sem, add=True)
            for i in range(8)
        ]
        for cp in cps:
            cp.wait()
        return carry

    lax.fori_loop(0, DEG_NBLK, blk, 0)
    plsc.subcore_barrier()
    obase = c * NP + s * 6400
    for k in range(6):
        pltpu.sync_copy(acc1.at[pl.ds(base + k * 1024, 1024)], stage)
        pltpu.sync_copy(stage, out.at[pl.ds(obase + k * 1024, 1024)])
    pltpu.sync_copy(acc1.at[pl.ds(base + 6144, 256)], stage.at[pl.ds(0, 256)])
    pltpu.sync_copy(stage.at[pl.ds(0, 256)], out.at[pl.ds(obase + 6144, 256)])


_deg_call = pl.kernel(
    _deg_body,
    out_type=jax.ShapeDtypeStruct((2 * NP,), F32),
    mesh=_mesh(),
    scratch_types=[
        pltpu.VMEM((8, 128), jnp.int32),
        pltpu.VMEM((128,), F32),
        pltpu.VMEM((1024,), F32),
        pltpu.VMEM((1024,), F32),
        pltpu.VMEM_SHARED((NP + 16,), F32),
        pltpu.SemaphoreType.DMA,
    ],
)


def _seg_body(src_hbm, dst_hbm, g_hbm, zrows, out, srcv, dstv1, csrc,
              cldst2, rows, zb, acc, sem):
    c = lax.axis_index("c")
    s = lax.axis_index("s")
    pltpu.sync_copy(zrows, zb)
    iota16 = lax.iota(jnp.int32, 16)
    padv = iota16 * 397 + s * 16

    def rnd(r, carry):
        lo = (2 * r + c) * CHUNK
        zbase = s * RPT
        for k in range(RPT // 16):
            pltpu.sync_copy(zb, acc.at[pl.ds(zbase + k * 16, 16)])
        plsc.subcore_barrier()
        dummy = CHUNK + s
        dummyv = jnp.zeros((16,), jnp.int32) + dummy

        def blk(b, carry2):
            eoff = (s * NBLK + b) * B
            pltpu.sync_copy(src_hbm.at[pl.ds(eoff, B)], srcv)
            pltpu.sync_copy(dst_hbm.at[pl.ds(eoff, B)], dstv1)

            def compact(j, cnt):
                d = dstv1[pl.ds(j * 16, 16)]
                sv = srcv[pl.ds(j * 16, 16)]
                m = (d >= lo) & (d < lo + CHUNK)
                cs = plsc.cumsum(m.astype(jnp.int32))
                pos = cnt + cs - 1
                plsc.store_scatter(csrc, [pos], sv, mask=m)
                plsc.store_scatter(
                    cldst2,
                    [lax.shift_right_logical(pos, 7), pos & 127],
                    d - lo, mask=m)
                return cnt + cs[15]

            cnt = lax.fori_loop(0, B // 16, compact, jnp.int32(0))
            for k in range(8):
                csrc[pl.ds(cnt + k * 16, 16)] = padv
                pp = cnt + k * 16 + iota16
                plsc.store_scatter(
                    cldst2,
                    [lax.shift_right_logical(pp, 7), pp & 127], dummyv)
            nseg = lax.shift_right_logical(cnt + 127, 7)

            def seg(k, carry3):
                pltpu.async_copy(
                    g_hbm.at[csrc.at[pl.ds(k * 128, 128)]], rows, sem
                ).wait()
                pltpu.async_copy(rows, acc.at[cldst2.at[k]], sem,
                                 add=True).wait()
                return carry3

            lax.fori_loop(0, nseg, seg, 0)
            return carry2

        lax.fori_loop(0, NBLK, blk, 0)
        plsc.subcore_barrier()
        wb = s * RPT
        for k in range(RPT // 128):
            pltpu.sync_copy(acc.at[pl.ds(wb + k * 128, 128)], rows)
            pltpu.sync_copy(rows, out.at[pl.ds(lo + wb + k * 128, 128)])
        plsc.subcore_barrier()
        return carry

    lax.fori_loop(0, NR, rnd, 0)


_seg_call = pl.kernel(
    _seg_body,
    out_type=jax.ShapeDtypeStruct((NP, W), F32),
    mesh=_mesh(),
    compiler_params=pltpu.CompilerParams(needs_layout_passes=False),
    scratch_types=[
        pltpu.VMEM((B,), jnp.int32),
        pltpu.VMEM((B,), jnp.int32),
        pltpu.VMEM((B + 128,), jnp.int32),
        pltpu.VMEM((B // 128 + 1, 128), jnp.int32),
        pltpu.VMEM((SEG, W), F32),
        pltpu.VMEM((16, W), F32),
        pltpu.VMEM_SHARED((CHUNK + 16, W), F32),
        pltpu.SemaphoreType.DMA,
    ],
)


def _tc1_body(x_ref, deg_ref, w1_ref, dinv_ref, g1_ref):
    deg = jnp.sum(deg_ref[...], axis=1, keepdims=True) + 1.0
    dinv = lax.rsqrt(deg)
    hw = jnp.dot(x_ref[...], w1_ref[...], preferred_element_type=F32)
    dinv_ref[...] = dinv
    g1_ref[...] = jnp.concatenate(
        [hw * dinv, jnp.zeros((BR, W - H), F32)], axis=1)


def _tc1(x_p, degT, w1p):
    return pl.pallas_call(
        _tc1_body,
        grid=(NBR,),
        in_specs=[
            pl.BlockSpec((BR, 8), lambda i: (i, 0)),
            pl.BlockSpec((BR, 8), lambda i: (i, 0)),
            pl.BlockSpec((8, H), lambda i: (0, 0)),
        ],
        out_specs=[
            pl.BlockSpec((BR, 1), lambda i: (i, 0)),
            pl.BlockSpec((BR, W), lambda i: (i, 0)),
        ],
        out_shape=[
            jax.ShapeDtypeStruct((NP, 1), F32),
            jax.ShapeDtypeStruct((NP, W), F32),
        ],
    )(x_p, degT, w1p)


def _tc2_body(s1_ref, g1_ref, dinv_ref, w2_ref, b1_ref, g2_ref):
    dinv = dinv_ref[...]
    h1 = jnp.maximum(
        dinv * (s1_ref[:, :H] + g1_ref[:, :H]) + b1_ref[...], 0.0)
    g2 = jnp.dot(h1, w2_ref[...], preferred_element_type=F32) * dinv
    g2_ref[...] = jnp.concatenate([g2, jnp.zeros((BR, W - H), F32)], axis=1)


def _tc2(s1, g1, dinv, w2, b1r):
    return pl.pallas_call(
        _tc2_body,
        grid=(NBR,),
        in_specs=[
            pl.BlockSpec((BR, W), lambda i: (i, 0)),
            pl.BlockSpec((BR, W), lambda i: (i, 0)),
            pl.BlockSpec((BR, 1), lambda i: (i, 0)),
            pl.BlockSpec((H, H), lambda i: (0, 0)),
            pl.BlockSpec((1, H), lambda i: (0, 0)),
        ],
        out_specs=pl.BlockSpec((BR, W), lambda i: (i, 0)),
        out_shape=jax.ShapeDtypeStruct((NP, W), F32),
    )(s1, g1, dinv, w2, b1r)


def _tc3_body(s2_ref, g2_ref, dinv_ref, x_ref, b2_ref, wp1a_ref, wp1b_ref,
              bp1_ref, wp2_ref, bp2_ref, out_ref, acc_h, acc_st):
    i = pl.program_id(0)

    @pl.when(i == 0)
    def _init():
        acc_h[...] = jnp.zeros((1, H), F32)
        acc_st[...] = jnp.zeros((1, 8), F32)

    rowid = lax.broadcasted_iota(jnp.int32, (BR, 1), 0) + i * BR
    valid = (rowid < N).astype(F32)
    h2 = jnp.maximum(
        dinv_ref[...] * (s2_ref[:, :H] + g2_ref[:, :H]) + b2_ref[...],
        0.0) * valid
    acc_h[...] += jnp.sum(h2, axis=0, keepdims=True)

    xb = x_ref[...] * valid
    x0 = xb[:, 0:1]
    x1 = xb[:, 1:2]
    x2 = xb[:, 2:3]
    x3 = xb[:, 3:4]
    x4 = xb[:, 4:5]
    m = jnp.where((x2 == 1.0) & (valid > 0.0), 1.0, 0.0)
    st = jnp.concatenate(
        [
            jnp.sum(x2, axis=0, keepdims=True),
            jnp.sum(x3, axis=0, keepdims=True),
            jnp.sum(x4, axis=0, keepdims=True),
            jnp.sum(x0 * m, axis=0, keepdims=True),
            jnp.sum(x1 * m, axis=0, keepdims=True),
            jnp.sum(m, axis=0, keepdims=True),
            jnp.zeros((1, 2), F32),
        ],
        axis=1,
    )
    acc_st[...] += st

    @pl.when(i == NBR - 1)
    def _fin():
        e = acc_h[...] * (1.0 / N)
        stv = acc_st[...]
        n_comp = stv[0, 0]
        n_and = stv[0, 1]
        n_or = stv[0, 2]
        sl = stv[0, 3]
        sm = stv[0, 4]
        cnt = stv[0, 5]
        has = cnt > 0.0
        cntc = jnp.maximum(cnt, 1.0)
        avg_l = jnp.where(has, sl / cntc, 0.0)
        avg_m = jnp.where(has, sm / cntc, 0.0)
        gf = jnp.concatenate(
            [
                n_comp.reshape(1, 1),
                n_and.reshape(1, 1),
                n_or.reshape(1, 1),
                (n_and + n_or).reshape(1, 1),
                avg_l.reshape(1, 1),
                avg_m.reshape(1, 1),
                jnp.full((1, 1), 0.2, F32),
                jnp.zeros((1, 1), F32),
            ],
            axis=1,
        )
        pre = (
            jnp.dot(e, wp1a_ref[...], preferred_element_type=F32)
            + jnp.dot(gf, wp1b_ref[...], preferred_element_type=F32)
            + bp1_ref[...]
        )
        hid = jnp.maximum(pre, 0.0)
        raw = jnp.dot(hid, wp2_ref[...], preferred_element_type=F32) + bp2_ref[...]
        out_ref[...] = 2.0 + jax.nn.sigmoid(raw) * 4.0


def _tc3(s2, g2, dinv, x_p, b2r, wp1a, wp1b, bp1r, wp2, bp2r):
    return pl.pallas_call(
        _tc3_body,
        grid=(NBR,),
        in_specs=[
            pl.BlockSpec((BR, W), lambda i: (i, 0)),
            pl.BlockSpec((BR, W), lambda i: (i, 0)),
            pl.BlockSpec((BR, 1), lambda i: (i, 0)),
            pl.BlockSpec((BR, 8), lambda i: (i, 0)),
            pl.BlockSpec((1, H), lambda i: (0, 0)),
            pl.BlockSpec((H, 32), lambda i: (0, 0)),
            pl.BlockSpec((8, 32), lambda i: (0, 0)),
            pl.BlockSpec((1, 32), lambda i: (0, 0)),
            pl.BlockSpec((32, 2), lambda i: (0, 0)),
            pl.BlockSpec((1, 2), lambda i: (0, 0)),
        ],
        out_specs=pl.BlockSpec((1, 2), lambda i: (0, 0)),
        out_shape=jax.ShapeDtypeStruct((1, 2), F32),
        scratch_shapes=[pltpu.VMEM((1, H), F32), pltpu.VMEM((1, 8), F32)],
    )(s2, g2, dinv, x_p, b2r, wp1a, wp1b, bp1r, wp2, bp2r)


def kernel(x, edge_index, W1, b1, W2, b2, Wp1, bp1, Wp2, bp2):
    src = edge_index[0].astype(jnp.int32)
    dst = edge_index[1].astype(jnp.int32)
    pad_src = jnp.arange(EPAD, dtype=jnp.int32) % N
    pad_dst = jnp.full((EPAD,), NP, jnp.int32)
    src_p = jnp.concatenate([src, pad_src])
    dst_p = jnp.concatenate([dst, pad_dst])
    dst2d = dst_p.reshape(EROWS, 128)
    zeros1 = jnp.zeros((1024,), F32)
    zrows = jnp.zeros((16, W), F32)

    deg_flat = _deg_call(dst2d, zeros1)
    degT = jnp.pad(deg_flat.reshape(2, NP).T, ((0, 0), (0, 6)))

    x_p = jnp.pad(x.astype(F32), ((0, NP - N), (0, 3)))
    w1p = jnp.pad(W1.astype(F32), ((0, 3), (0, 0)))
    dinv, g1 = _tc1(x_p, degT, w1p)

    s1 = _seg_call(src_p, dst_p, g1, zrows)
    g2 = _tc2(s1, g1, dinv, W2.astype(F32), b1.reshape(1, H).astype(F32))
    s2 = _seg_call(src_p, dst_p, g2, zrows)

    wp1 = Wp1.astype(F32)
    wp1a = wp1[:H]
    wp1b = jnp.pad(wp1[H:], ((0, 1), (0, 0)))
    out = _tc3(
        s2, g2, dinv, x_p,
        b2.reshape(1, H).astype(F32),
        wp1a, wp1b,
        bp1.reshape(1, 32).astype(F32),
        Wp2.astype(F32),
        bp2.reshape(1, 2).astype(F32),
    )
    return out


# speedup vs baseline: 11.1914x; 1.0106x over previous
import functools

import jax


def _dyn_gather(x, idx):
    dnums = jax.lax.GatherDimensionNumbers(
        offset_dims=(), collapsed_slice_dims=(0,), start_index_map=(0,))
    return jax.lax.gather(
        x, idx[:, None], dnums, slice_sizes=(1,),
        mode=jax.lax.GatherScatterMode.PROMISE_IN_BOUNDS)
import jax.numpy as jnp
from jax import lax
from jax.experimental import pallas as pl
from jax.experimental.pallas import tpu as pltpu
from jax.experimental.pallas import tpu_sc as plsc

N = 100000
E = 3200000
H = 64
W = 128
CHUNK = 12800
NCH = 8
NR = NCH // 2
NP = CHUNK * NCH
B = 2048
SEG = 128
NSEG = B // SEG
NBLK = 98
EP = 16 * B * NBLK
EPAD = EP - E
EROWS = EP // 128
DEG_NBLK = 98
RPT = CHUNK // 16
BR = 2048
NBR = NP // BR
F32 = jnp.float32


def _mesh():
    return plsc.VectorSubcoreMesh(
        core_axis_name="c", subcore_axis_name="s", num_cores=2, num_subcores=16
    )


def _deg_body(dst2d, zeros1, out, dstv, onesv, zb1, stage, acc1, sem):
    c = lax.axis_index("c")
    s = lax.axis_index("s")
    w = c * 16 + s
    pltpu.sync_copy(zeros1, zb1)
    for k in range(8):
        onesv[pl.ds(k * 16, 16)] = jnp.ones((16,), F32)
    base = s * 6400
    for k in range(6):
        pltpu.sync_copy(zb1, acc1.at[pl.ds(base + k * 1024, 1024)])
    pltpu.sync_copy(zb1.at[pl.ds(0, 256)], acc1.at[pl.ds(base + 6144, 256)])
    plsc.subcore_barrier()

    def blk(b, carry):
        roff = w * (DEG_NBLK * 8) + b * 8
        pltpu.sync_copy(dst2d.at[pl.ds(roff, 8)], dstv)
        cps = [
            pltpu.async_copy(onesv, acc1.at[dstv.at[i]], sem, add=True)
            for i in range(8)
        ]
        for cp in cps:
            cp.wait()
        return carry

    lax.fori_loop(0, DEG_NBLK, blk, 0)
    plsc.subcore_barrier()
    obase = c * NP + s * 6400
    for k in range(6):
        pltpu.sync_copy(acc1.at[pl.ds(base + k * 1024, 1024)], stage)
        pltpu.sync_copy(stage, out.at[pl.ds(obase + k * 1024, 1024)])
    pltpu.sync_copy(acc1.at[pl.ds(base + 6144, 256)], stage.at[pl.ds(0, 256)])
    pltpu.sync_copy(stage.at[pl.ds(0, 256)], out.at[pl.ds(obase + 6144, 256)])


_deg_call = pl.kernel(
    _deg_body,
    out_type=jax.ShapeDtypeStruct((2 * NP,), F32),
    mesh=_mesh(),
    scratch_types=[
        pltpu.VMEM((8, 128), jnp.int32),
        pltpu.VMEM((128,), F32),
        pltpu.VMEM((1024,), F32),
        pltpu.VMEM((1024,), F32),
        pltpu.VMEM_SHARED((NP + 16,), F32),
        pltpu.SemaphoreType.DMA,
    ],
)


def _seg_body(src_hbm, dst_hbm, g_hbm, zrows, out, srcv, dstv1, csrc,
              cldst2, rows, zb, acc, sem):
    c = lax.axis_index("c")
    s = lax.axis_index("s")
    pltpu.sync_copy(zrows, zb)
    iota16 = lax.iota(jnp.int32, 16)
    padv = iota16 * 397 + s * 16

    def rnd(r, carry):
        lo = (2 * r + c) * CHUNK
        zbase = s * RPT
        for k in range(RPT // 16):
            pltpu.sync_copy(zb, acc.at[pl.ds(zbase + k * 16, 16)])
        plsc.subcore_barrier()
        dummy = CHUNK + s
        dummyv = jnp.zeros((16,), jnp.int32) + dummy

        def blk(b, carry2):
            eoff = (s * NBLK + b) * B
            pltpu.sync_copy(src_hbm.at[pl.ds(eoff, B)], srcv)
            pltpu.sync_copy(dst_hbm.at[pl.ds(eoff, B)], dstv1)

            def compact(j, cnt):
                for u in range(2):
                    off = j * 32 + u * 16
                    d = dstv1[pl.ds(off, 16)]
                    sv = srcv[pl.ds(off, 16)]
                    m = (d >= lo) & (d < lo + CHUNK)
                    cs = plsc.cumsum(m.astype(jnp.int32))
                    pos = cnt + cs - 1
                    plsc.store_scatter(csrc, [pos], sv, mask=m)
                    plsc.store_scatter(
                        cldst2,
                        [lax.shift_right_logical(pos, 7), pos & 127],
                        d - lo, mask=m)
                    cnt = cnt + cs[15]
                return cnt

            cnt = lax.fori_loop(0, B // 32, compact, jnp.int32(0))
            for k in range(8):
                csrc[pl.ds(cnt + k * 16, 16)] = padv
                pp = cnt + k * 16 + iota16
                plsc.store_scatter(
                    cldst2,
                    [lax.shift_right_logical(pp, 7), pp & 127], dummyv)
            nseg = lax.shift_right_logical(cnt + 127, 7)

            def seg(k, carry3):
                pltpu.async_copy(
                    g_hbm.at[csrc.at[pl.ds(k * 128, 128)]], rows, sem
                ).wait()
                pltpu.async_copy(rows, acc.at[cldst2.at[k]], sem,
                                 add=True).wait()
                return carry3

            lax.fori_loop(0, nseg, seg, 0)
            return carry2

        lax.fori_loop(0, NBLK, blk, 0)
        plsc.subcore_barrier()
        wb = s * RPT
        for k in range(RPT // 128):
            pltpu.sync_copy(acc.at[pl.ds(wb + k * 128, 128)], rows)
            pltpu.sync_copy(rows, out.at[pl.ds(lo + wb + k * 128, 128)])
        if RPT % 128:
            t = RPT % 128
            b0 = wb + RPT - t
            pltpu.sync_copy(acc.at[pl.ds(b0, t)], rows.at[pl.ds(0, t)])
            pltpu.sync_copy(rows.at[pl.ds(0, t)], out.at[pl.ds(lo + b0, t)])
        plsc.subcore_barrier()
        return carry

    lax.fori_loop(0, NR, rnd, 0)


_seg_call = pl.kernel(
    _seg_body,
    out_type=jax.ShapeDtypeStruct((NP, W), F32),
    mesh=_mesh(),
    compiler_params=pltpu.CompilerParams(needs_layout_passes=False),
    scratch_types=[
        pltpu.VMEM((B,), jnp.int32),
        pltpu.VMEM((B,), jnp.int32),
        pltpu.VMEM((B + 128,), jnp.int32),
        pltpu.VMEM((B // 128 + 1, 128), jnp.int32),
        pltpu.VMEM((SEG, W), F32),
        pltpu.VMEM((16, W), F32),
        pltpu.VMEM_SHARED((CHUNK + 16, W), F32),
        pltpu.SemaphoreType.DMA,
    ],
)


def _tc1_body(x_ref, deg_ref, w1_ref, dinv_ref, g1_ref):
    deg = jnp.sum(deg_ref[...], axis=1, keepdims=True) + 1.0
    dinv = lax.rsqrt(deg)
    hw = jnp.dot(x_ref[...], w1_ref[...], preferred_element_type=F32)
    dinv_ref[...] = dinv
    g1_ref[...] = jnp.concatenate(
        [hw * dinv, jnp.zeros((BR, W - H), F32)], axis=1)


def _tc1(x_p, degT, w1p):
    return pl.pallas_call(
        _tc1_body,
        grid=(NBR,),
        in_specs=[
            pl.BlockSpec((BR, 8), lambda i: (i, 0)),
            pl.BlockSpec((BR, 8), lambda i: (i, 0)),
            pl.BlockSpec((8, H), lambda i: (0, 0)),
        ],
        out_specs=[
            pl.BlockSpec((BR, 1), lambda i: (i, 0)),
            pl.BlockSpec((BR, W), lambda i: (i, 0)),
        ],
        out_shape=[
            jax.ShapeDtypeStruct((NP, 1), F32),
            jax.ShapeDtypeStruct((NP, W), F32),
        ],
    )(x_p, degT, w1p)


def _tc2_body(s1_ref, g1_ref, dinv_ref, w2_ref, b1_ref, g2_ref):
    dinv = dinv_ref[...]
    h1 = jnp.maximum(
        dinv * (s1_ref[:, :H] + g1_ref[:, :H]) + b1_ref[...], 0.0)
    g2 = jnp.dot(h1, w2_ref[...], preferred_element_type=F32) * dinv
    g2_ref[...] = jnp.concatenate([g2, jnp.zeros((BR, W - H), F32)], axis=1)


def _tc2(s1, g1, dinv, w2, b1r):
    return pl.pallas_call(
        _tc2_body,
        grid=(NBR,),
        in_specs=[
            pl.BlockSpec((BR, W), lambda i: (i, 0)),
            pl.BlockSpec((BR, W), lambda i: (i, 0)),
            pl.BlockSpec((BR, 1), lambda i: (i, 0)),
            pl.BlockSpec((H, H), lambda i: (0, 0)),
            pl.BlockSpec((1, H), lambda i: (0, 0)),
        ],
        out_specs=pl.BlockSpec((BR, W), lambda i: (i, 0)),
        out_shape=jax.ShapeDtypeStruct((NP, W), F32),
    )(s1, g1, dinv, w2, b1r)


def _tc3_body(s2_ref, g2_ref, dinv_ref, x_ref, b2_ref, wp1a_ref, wp1b_ref,
              bp1_ref, wp2_ref, bp2_ref, out_ref, acc_h, acc_st):
    i = pl.program_id(0)

    @pl.when(i == 0)
    def _init():
        acc_h[...] = jnp.zeros((1, H), F32)
        acc_st[...] = jnp.zeros((1, 8), F32)

    rowid = lax.broadcasted_iota(jnp.int32, (BR, 1), 0) + i * BR
    valid = (rowid < N).astype(F32)
    h2 = jnp.maximum(
        dinv_ref[...] * (s2_ref[:, :H] + g2_ref[:, :H]) + b2_ref[...],
        0.0) * valid
    acc_h[...] += jnp.sum(h2, axis=0, keepdims=True)

    xb = x_ref[...] * valid
    x0 = xb[:, 0:1]
    x1 = xb[:, 1:2]
    x2 = xb[:, 2:3]
    x3 = xb[:, 3:4]
    x4 = xb[:, 4:5]
    m = jnp.where((x2 == 1.0) & (valid > 0.0), 1.0, 0.0)
    st = jnp.concatenate(
        [
            jnp.sum(x2, axis=0, keepdims=True),
            jnp.sum(x3, axis=0, keepdims=True),
            jnp.sum(x4, axis=0, keepdims=True),
            jnp.sum(x0 * m, axis=0, keepdims=True),
            jnp.sum(x1 * m, axis=0, keepdims=True),
            jnp.sum(m, axis=0, keepdims=True),
            jnp.zeros((1, 2), F32),
        ],
        axis=1,
    )
    acc_st[...] += st

    @pl.when(i == NBR - 1)
    def _fin():
        e = acc_h[...] * (1.0 / N)
        stv = acc_st[...]
        n_comp = stv[0, 0]
        n_and = stv[0, 1]
        n_or = stv[0, 2]
        sl = stv[0, 3]
        sm = stv[0, 4]
        cnt = stv[0, 5]
        has = cnt > 0.0
        cntc = jnp.maximum(cnt, 1.0)
        avg_l = jnp.where(has, sl / cntc, 0.0)
        avg_m = jnp.where(has, sm / cntc, 0.0)
        gf = jnp.concatenate(
            [
                n_comp.reshape(1, 1),
                n_and.reshape(1, 1),
                n_or.reshape(1, 1),
                (n_and + n_or).reshape(1, 1),
                avg_l.reshape(1, 1),
                avg_m.reshape(1, 1),
                jnp.full((1, 1), 0.2, F32),
                jnp.zeros((1, 1), F32),
            ],
            axis=1,
        )
        pre = (
            jnp.dot(e, wp1a_ref[...], preferred_element_type=F32)
            + jnp.dot(gf, wp1b_ref[...], preferred_element_type=F32)
            + bp1_ref[...]
        )
        hid = jnp.maximum(pre, 0.0)
        raw = jnp.dot(hid, wp2_ref[...], preferred_element_type=F32) + bp2_ref[...]
        out_ref[...] = 2.0 + jax.nn.sigmoid(raw) * 4.0


def _tc3(s2, g2, dinv, x_p, b2r, wp1a, wp1b, bp1r, wp2, bp2r):
    return pl.pallas_call(
        _tc3_body,
        grid=(NBR,),
        in_specs=[
            pl.BlockSpec((BR, W), lambda i: (i, 0)),
            pl.BlockSpec((BR, W), lambda i: (i, 0)),
            pl.BlockSpec((BR, 1), lambda i: (i, 0)),
            pl.BlockSpec((BR, 8), lambda i: (i, 0)),
            pl.BlockSpec((1, H), lambda i: (0, 0)),
            pl.BlockSpec((H, 32), lambda i: (0, 0)),
            pl.BlockSpec((8, 32), lambda i: (0, 0)),
            pl.BlockSpec((1, 32), lambda i: (0, 0)),
            pl.BlockSpec((32, 2), lambda i: (0, 0)),
            pl.BlockSpec((1, 2), lambda i: (0, 0)),
        ],
        out_specs=pl.BlockSpec((1, 2), lambda i: (0, 0)),
        out_shape=jax.ShapeDtypeStruct((1, 2), F32),
        scratch_shapes=[pltpu.VMEM((1, H), F32), pltpu.VMEM((1, 8), F32)],
    )(s2, g2, dinv, x_p, b2r, wp1a, wp1b, bp1r, wp2, bp2r)


def kernel(x, edge_index, W1, b1, W2, b2, Wp1, bp1, Wp2, bp2):
    src = edge_index[0].astype(jnp.int32)
    dst = edge_index[1].astype(jnp.int32)
    pad_src = jnp.arange(EPAD, dtype=jnp.int32) % N
    pad_dst = jnp.full((EPAD,), NP, jnp.int32)
    src_p = jnp.concatenate([src, pad_src])
    dst_p = jnp.concatenate([dst, pad_dst])
    dst2d = dst_p.reshape(EROWS, 128)
    zeros1 = jnp.zeros((1024,), F32)
    zrows = jnp.zeros((16, W), F32)

    deg_flat = _deg_call(dst2d, zeros1)
    degT = jnp.pad(deg_flat.reshape(2, NP).T, ((0, 0), (0, 6)))

    x_p = jnp.pad(x.astype(F32), ((0, NP - N), (0, 3)))
    w1p = jnp.pad(W1.astype(F32), ((0, 3), (0, 0)))
    dinv, g1 = _tc1(x_p, degT, w1p)

    s1 = _seg_call(src_p, dst_p, g1, zrows)
    g2 = _tc2(s1, g1, dinv, W2.astype(F32), b1.reshape(1, H).astype(F32))
    s2 = _seg_call(src_p, dst_p, g2, zrows)

    wp1 = Wp1.astype(F32)
    wp1a = wp1[:H]
    wp1b = jnp.pad(wp1[H:], ((0, 1), (0, 0)))
    out = _tc3(
        s2, g2, dinv, x_p,
        b2.reshape(1, H).astype(F32),
        wp1a, wp1b,
        bp1.reshape(1, 32).astype(F32),
        Wp2.astype(F32),
        bp2.reshape(1, 2).astype(F32),
    )
    return out


# speedup vs baseline: 13.7998x; 1.2331x over previous
import functools

import jax


def _dyn_gather(x, idx):
    dnums = jax.lax.GatherDimensionNumbers(
        offset_dims=(), collapsed_slice_dims=(0,), start_index_map=(0,))
    return jax.lax.gather(
        x, idx[:, None], dnums, slice_sizes=(1,),
        mode=jax.lax.GatherScatterMode.PROMISE_IN_BOUNDS)
import jax.numpy as jnp
from jax import lax
from jax.experimental import pallas as pl
from jax.experimental.pallas import tpu as pltpu
from jax.experimental.pallas import tpu_sc as plsc

N = 100000
E = 3200000
H = 64
W = 128
CHUNK = 10240
NCH = 10
NR = NCH // 2
NP = CHUNK * NCH
B = 2048
SEG = 128
NSEG = B // SEG
NBLK = 98
EP = 16 * B * NBLK
EPAD = EP - E
EROWS = EP // 128
DEG_NBLK = 98
RPT = CHUNK // 16
BR = 2048
NBR = NP // BR
F32 = jnp.float32


def _mesh():
    return plsc.VectorSubcoreMesh(
        core_axis_name="c", subcore_axis_name="s", num_cores=2, num_subcores=16
    )


def _deg_body(dst2d, zeros1, out, dstv, onesv, zb1, stage, acc1, sem):
    c = lax.axis_index("c")
    s = lax.axis_index("s")
    w = c * 16 + s
    pltpu.sync_copy(zeros1, zb1)
    for k in range(8):
        onesv[pl.ds(k * 16, 16)] = jnp.ones((16,), F32)
    base = s * 6400
    for k in range(6):
        pltpu.sync_copy(zb1, acc1.at[pl.ds(base + k * 1024, 1024)])
    pltpu.sync_copy(zb1.at[pl.ds(0, 256)], acc1.at[pl.ds(base + 6144, 256)])
    plsc.subcore_barrier()

    def blk(b, carry):
        roff = w * (DEG_NBLK * 8) + b * 8
        pltpu.sync_copy(dst2d.at[pl.ds(roff, 8)], dstv)
        cps = [
            pltpu.async_copy(onesv, acc1.at[dstv.at[i]], sem, add=True)
            for i in range(8)
        ]
        for cp in cps:
            cp.wait()
        return carry

    lax.fori_loop(0, DEG_NBLK, blk, 0)
    plsc.subcore_barrier()
    obase = c * NP + s * 6400
    for k in range(6):
        pltpu.sync_copy(acc1.at[pl.ds(base + k * 1024, 1024)], stage)
        pltpu.sync_copy(stage, out.at[pl.ds(obase + k * 1024, 1024)])
    pltpu.sync_copy(acc1.at[pl.ds(base + 6144, 256)], stage.at[pl.ds(0, 256)])
    pltpu.sync_copy(stage.at[pl.ds(0, 256)], out.at[pl.ds(obase + 6144, 256)])


_deg_call = pl.kernel(
    _deg_body,
    out_type=jax.ShapeDtypeStruct((2 * NP,), F32),
    mesh=_mesh(),
    scratch_types=[
        pltpu.VMEM((8, 128), jnp.int32),
        pltpu.VMEM((128,), F32),
        pltpu.VMEM((1024,), F32),
        pltpu.VMEM((1024,), F32),
        pltpu.VMEM_SHARED((NP + 16,), F32),
        pltpu.SemaphoreType.DMA,
    ],
)


def _seg_body(src_hbm, dst_hbm, g_hbm, zrows, out, srcv, dstv1, csrc,
              cldst2, rows, zb, acc, semL, semG, semS):
    c = lax.axis_index("c")
    s = lax.axis_index("s")
    pltpu.sync_copy(zrows, zb)
    iota16 = lax.iota(jnp.int32, 16)
    padv = iota16 * 397 + s * 16

    def rnd(r, carry):
        lo = (2 * r + c) * CHUNK
        zbase = s * RPT
        for k in range(RPT // 16):
            pltpu.sync_copy(zb, acc.at[pl.ds(zbase + k * 16, 16)])
        plsc.subcore_barrier()
        dummy = CHUNK + s
        dummyv = jnp.zeros((16,), jnp.int32) + dummy

        def blk(b, carry2):
            eoff = (s * NBLK + b) * B
            cpS = pltpu.async_copy(src_hbm.at[pl.ds(eoff, B)], srcv, semL)
            cpD = pltpu.async_copy(dst_hbm.at[pl.ds(eoff, B)], dstv1, semL)
            cpS.wait()
            cpD.wait()

            def compact(j, cnt):
                for u in range(2):
                    off = j * 32 + u * 16
                    d = dstv1[pl.ds(off, 16)]
                    sv = srcv[pl.ds(off, 16)]
                    m = (d >= lo) & (d < lo + CHUNK)
                    cs = plsc.cumsum(m.astype(jnp.int32))
                    pos = cnt + cs - 1
                    plsc.store_scatter(csrc, [pos], sv, mask=m)
                    plsc.store_scatter(
                        cldst2,
                        [lax.shift_right_logical(pos, 7), pos & 127],
                        d - lo, mask=m)
                    cnt = cnt + cs[15]
                return cnt

            cnt = lax.fori_loop(0, B // 32, compact, jnp.int32(0))
            for k in range(8):
                csrc[pl.ds(cnt + k * 16, 16)] = padv
                pp = cnt + k * 16 + iota16
                plsc.store_scatter(
                    cldst2,
                    [lax.shift_right_logical(pp, 7), pp & 127], dummyv)
            nseg = lax.shift_right_logical(cnt + 127, 7)

            @pl.when(nseg > 0)
            def _pro():
                pltpu.async_copy(
                    g_hbm.at[csrc.at[pl.ds(0, 128)]],
                    rows.at[pl.ds(0, 128)], semG.at[0])

            def seg(k, carry3):
                kp = k % 2

                @pl.when(k >= 1)
                def _ws():
                    pltpu.make_async_copy(
                        rows.at[pl.ds(0, 128)], acc.at[pl.ds(0, 128)],
                        semS.at[1 - kp]).wait()

                @pl.when(k + 1 < nseg)
                def _fg():
                    pltpu.async_copy(
                        g_hbm.at[csrc.at[pl.ds((k + 1) * 128, 128)]],
                        rows.at[pl.ds((1 - kp) * 128, 128)],
                        semG.at[1 - kp])

                pltpu.make_async_copy(
                    g_hbm.at[pl.ds(0, 128)], rows.at[pl.ds(0, 128)],
                    semG.at[kp]).wait()
                pltpu.async_copy(
                    rows.at[pl.ds(kp * 128, 128)], acc.at[cldst2.at[k]],
                    semS.at[kp], add=True)
                return carry3

            lax.fori_loop(0, nseg, seg, 0)

            @pl.when(nseg > 0)
            def _epi():
                pltpu.make_async_copy(
                    rows.at[pl.ds(0, 128)], acc.at[pl.ds(0, 128)],
                    semS.at[(nseg - 1) % 2]).wait()
            return carry2

        lax.fori_loop(0, NBLK, blk, 0)
        plsc.subcore_barrier()
        wb = s * RPT
        for k in range(RPT // 128):
            pltpu.sync_copy(acc.at[pl.ds(wb + k * 128, 128)],
                            rows.at[pl.ds(0, 128)])
            pltpu.sync_copy(rows.at[pl.ds(0, 128)],
                            out.at[pl.ds(lo + wb + k * 128, 128)])
        if RPT % 128:
            t = RPT % 128
            b0 = wb + RPT - t
            pltpu.sync_copy(acc.at[pl.ds(b0, t)], rows.at[pl.ds(0, t)])
            pltpu.sync_copy(rows.at[pl.ds(0, t)], out.at[pl.ds(lo + b0, t)])
        plsc.subcore_barrier()
        return carry

    lax.fori_loop(0, NR, rnd, 0)


_seg_call = pl.kernel(
    _seg_body,
    out_type=jax.ShapeDtypeStruct((NP, W), F32),
    mesh=_mesh(),
    compiler_params=pltpu.CompilerParams(needs_layout_passes=False),
    scratch_types=[
        pltpu.VMEM((B,), jnp.int32),
        pltpu.VMEM((B,), jnp.int32),
        pltpu.VMEM((B + 128,), jnp.int32),
        pltpu.VMEM((B // 128 + 1, 128), jnp.int32),
        pltpu.VMEM((2 * SEG, W), F32),
        pltpu.VMEM((16, W), F32),
        pltpu.VMEM_SHARED((CHUNK + 16, W), F32),
        pltpu.SemaphoreType.DMA,
        pltpu.SemaphoreType.DMA((2,)),
        pltpu.SemaphoreType.DMA((2,)),
    ],
)


def _tc1_body(x_ref, deg_ref, w1_ref, dinv_ref, g1_ref):
    deg = jnp.sum(deg_ref[...], axis=1, keepdims=True) + 1.0
    dinv = lax.rsqrt(deg)
    hw = jnp.dot(x_ref[...], w1_ref[...], preferred_element_type=F32)
    dinv_ref[...] = dinv
    g1_ref[...] = jnp.concatenate(
        [hw * dinv, jnp.zeros((BR, W - H), F32)], axis=1)


def _tc1(x_p, degT, w1p):
    return pl.pallas_call(
        _tc1_body,
        grid=(NBR,),
        in_specs=[
            pl.BlockSpec((BR, 8), lambda i: (i, 0)),
            pl.BlockSpec((BR, 8), lambda i: (i, 0)),
            pl.BlockSpec((8, H), lambda i: (0, 0)),
        ],
        out_specs=[
            pl.BlockSpec((BR, 1), lambda i: (i, 0)),
            pl.BlockSpec((BR, W), lambda i: (i, 0)),
        ],
        out_shape=[
            jax.ShapeDtypeStruct((NP, 1), F32),
            jax.ShapeDtypeStruct((NP, W), F32),
        ],
    )(x_p, degT, w1p)


def _tc2_body(s1_ref, g1_ref, dinv_ref, w2_ref, b1_ref, g2_ref):
    dinv = dinv_ref[...]
    h1 = jnp.maximum(
        dinv * (s1_ref[:, :H] + g1_ref[:, :H]) + b1_ref[...], 0.0)
    g2 = jnp.dot(h1, w2_ref[...], preferred_element_type=F32) * dinv
    g2_ref[...] = jnp.concatenate([g2, jnp.zeros((BR, W - H), F32)], axis=1)


def _tc2(s1, g1, dinv, w2, b1r):
    return pl.pallas_call(
        _tc2_body,
        grid=(NBR,),
        in_specs=[
            pl.BlockSpec((BR, W), lambda i: (i, 0)),
            pl.BlockSpec((BR, W), lambda i: (i, 0)),
            pl.BlockSpec((BR, 1), lambda i: (i, 0)),
            pl.BlockSpec((H, H), lambda i: (0, 0)),
            pl.BlockSpec((1, H), lambda i: (0, 0)),
        ],
        out_specs=pl.BlockSpec((BR, W), lambda i: (i, 0)),
        out_shape=jax.ShapeDtypeStruct((NP, W), F32),
    )(s1, g1, dinv, w2, b1r)


def _tc3_body(s2_ref, g2_ref, dinv_ref, x_ref, b2_ref, wp1a_ref, wp1b_ref,
              bp1_ref, wp2_ref, bp2_ref, out_ref, acc_h, acc_st):
    i = pl.program_id(0)

    @pl.when(i == 0)
    def _init():
        acc_h[...] = jnp.zeros((1, H), F32)
        acc_st[...] = jnp.zeros((1, 8), F32)

    rowid = lax.broadcasted_iota(jnp.int32, (BR, 1), 0) + i * BR
    valid = (rowid < N).astype(F32)
    h2 = jnp.maximum(
        dinv_ref[...] * (s2_ref[:, :H] + g2_ref[:, :H]) + b2_ref[...],
        0.0) * valid
    acc_h[...] += jnp.sum(h2, axis=0, keepdims=True)

    xb = x_ref[...] * valid
    x0 = xb[:, 0:1]
    x1 = xb[:, 1:2]
    x2 = xb[:, 2:3]
    x3 = xb[:, 3:4]
    x4 = xb[:, 4:5]
    m = jnp.where((x2 == 1.0) & (valid > 0.0), 1.0, 0.0)
    st = jnp.concatenate(
        [
            jnp.sum(x2, axis=0, keepdims=True),
            jnp.sum(x3, axis=0, keepdims=True),
            jnp.sum(x4, axis=0, keepdims=True),
            jnp.sum(x0 * m, axis=0, keepdims=True),
            jnp.sum(x1 * m, axis=0, keepdims=True),
            jnp.sum(m, axis=0, keepdims=True),
            jnp.zeros((1, 2), F32),
        ],
        axis=1,
    )
    acc_st[...] += st

    @pl.when(i == NBR - 1)
    def _fin():
        e = acc_h[...] * (1.0 / N)
        stv = acc_st[...]
        n_comp = stv[0, 0]
        n_and = stv[0, 1]
        n_or = stv[0, 2]
        sl = stv[0, 3]
        sm = stv[0, 4]
        cnt = stv[0, 5]
        has = cnt > 0.0
        cntc = jnp.maximum(cnt, 1.0)
        avg_l = jnp.where(has, sl / cntc, 0.0)
        avg_m = jnp.where(has, sm / cntc, 0.0)
        gf = jnp.concatenate(
            [
                n_comp.reshape(1, 1),
                n_and.reshape(1, 1),
                n_or.reshape(1, 1),
                (n_and + n_or).reshape(1, 1),
                avg_l.reshape(1, 1),
                avg_m.reshape(1, 1),
                jnp.full((1, 1), 0.2, F32),
                jnp.zeros((1, 1), F32),
            ],
            axis=1,
        )
        pre = (
            jnp.dot(e, wp1a_ref[...], preferred_element_type=F32)
            + jnp.dot(gf, wp1b_ref[...], preferred_element_type=F32)
            + bp1_ref[...]
        )
        hid = jnp.maximum(pre, 0.0)
        raw = jnp.dot(hid, wp2_ref[...], preferred_element_type=F32) + bp2_ref[...]
        out_ref[...] = 2.0 + jax.nn.sigmoid(raw) * 4.0


def _tc3(s2, g2, dinv, x_p, b2r, wp1a, wp1b, bp1r, wp2, bp2r):
    return pl.pallas_call(
        _tc3_body,
        grid=(NBR,),
        in_specs=[
            pl.BlockSpec((BR, W), lambda i: (i, 0)),
            pl.BlockSpec((BR, W), lambda i: (i, 0)),
            pl.BlockSpec((BR, 1), lambda i: (i, 0)),
            pl.BlockSpec((BR, 8), lambda i: (i, 0)),
            pl.BlockSpec((1, H), lambda i: (0, 0)),
            pl.BlockSpec((H, 32), lambda i: (0, 0)),
            pl.BlockSpec((8, 32), lambda i: (0, 0)),
            pl.BlockSpec((1, 32), lambda i: (0, 0)),
            pl.BlockSpec((32, 2), lambda i: (0, 0)),
            pl.BlockSpec((1, 2), lambda i: (0, 0)),
        ],
        out_specs=pl.BlockSpec((1, 2), lambda i: (0, 0)),
        out_shape=jax.ShapeDtypeStruct((1, 2), F32),
        scratch_shapes=[pltpu.VMEM((1, H), F32), pltpu.VMEM((1, 8), F32)],
    )(s2, g2, dinv, x_p, b2r, wp1a, wp1b, bp1r, wp2, bp2r)


def kernel(x, edge_index, W1, b1, W2, b2, Wp1, bp1, Wp2, bp2):
    src = edge_index[0].astype(jnp.int32)
    dst = edge_index[1].astype(jnp.int32)
    pad_src = jnp.arange(EPAD, dtype=jnp.int32) % N
    pad_dst = jnp.full((EPAD,), NP, jnp.int32)
    src_p = jnp.concatenate([src, pad_src])
    dst_p = jnp.concatenate([dst, pad_dst])
    dst2d = dst_p.reshape(EROWS, 128)
    zeros1 = jnp.zeros((1024,), F32)
    zrows = jnp.zeros((16, W), F32)

    deg_flat = _deg_call(dst2d, zeros1)
    degT = jnp.pad(deg_flat.reshape(2, NP).T, ((0, 0), (0, 6)))

    x_p = jnp.pad(x.astype(F32), ((0, NP - N), (0, 3)))
    w1p = jnp.pad(W1.astype(F32), ((0, 3), (0, 0)))
    dinv, g1 = _tc1(x_p, degT, w1p)

    s1 = _seg_call(src_p, dst_p, g1, zrows)
    g2 = _tc2(s1, g1, dinv, W2.astype(F32), b1.reshape(1, H).astype(F32))
    s2 = _seg_call(src_p, dst_p, g2, zrows)

    wp1 = Wp1.astype(F32)
    wp1a = wp1[:H]
    wp1b = jnp.pad(wp1[H:], ((0, 1), (0, 0)))
    out = _tc3(
        s2, g2, dinv, x_p,
        b2.reshape(1, H).astype(F32),
        wp1a, wp1b,
        bp1.reshape(1, 32).astype(F32),
        Wp2.astype(F32),
        bp2.reshape(1, 2).astype(F32),
    )
    return out
